# baseline (speedup 1.0000x reference)
import jax
import jax.numpy as jnp
import jax.experimental.pallas as pl

B, S, D = 4, 8192, 2048
CAP = 2048
TB = 1024
G = D // 8
K_ALIGN = 27


def _matvec_kernel(xs_ref, ws_ref, o_ref):
    v4 = []
    for half in range(2):
        ps = []
        for i in range(4 * half, 4 * half + 4):
            a = xs_ref[i, 0].astype(jnp.float32)
            ps.append(a * ws_ref[i][None, :].astype(jnp.float32))
        amax = jnp.maximum(jnp.maximum(jnp.abs(ps[0]), jnp.abs(ps[1])),
                           jnp.maximum(jnp.abs(ps[2]), jnp.abs(ps[3])))
        bits = jax.lax.bitcast_convert_type(amax, jnp.int32)
        eb = jnp.clip(bits >> 23, 64, 254)
        graninv = jax.lax.bitcast_convert_type((281 - eb) << 23, jnp.float32)
        gran = jax.lax.bitcast_convert_type((eb - K_ALIGN) << 23, jnp.float32)
        s = ((ps[0] * graninv).astype(jnp.int32)
             + (ps[1] * graninv).astype(jnp.int32)
             + (ps[2] * graninv).astype(jnp.int32)
             + (ps[3] * graninv).astype(jnp.int32))
        v4.append(s.astype(jnp.float32) * gran)
    g8 = v4[0] + v4[1]

    zpad = jnp.zeros((TB, 240), dtype=jnp.float32)
    gp = jnp.concatenate([g8, zpad], axis=1)
    acc = g8
    for k in range(1, 16):
        acc = acc + jax.lax.slice(gp, (0, k), (TB, k + G))
    accp = jnp.concatenate([acc, zpad], axis=1)
    tot = acc
    for k in range(1, 16):
        tot = tot + jax.lax.slice(accp, (0, 16 * k), (TB, 16 * k + G))
    o_ref[0, 0] = tot[:, 0]


def _topk_kernel(l_ref, o_ref):
    l = l_ref[...]
    m = jnp.max(l, axis=1, keepdims=True)
    key = jnp.exp(l - m)
    idx = jax.lax.broadcasted_iota(jnp.int32, (B, S), 1)
    lane = jax.lax.broadcasted_iota(jnp.int32, (B, S), 1)

    def rolled(a, j):
        left = jnp.concatenate([a[:, j:], a[:, :j]], axis=1)
        right = jnp.concatenate([a[:, -j:], a[:, :-j]], axis=1)
        return left, right

    k = 2
    while k <= S:
        j = k // 2
        while j >= 1:
            is_low = (lane & j) == 0
            asc = (lane & k) == 0
            kl, kr = rolled(key, j)
            il, ir = rolled(idx, j)
            pk = jnp.where(is_low, kl, kr)
            pi = jnp.where(is_low, il, ir)
            self_first = (key > pk) | ((key == pk) & (idx < pi))
            keep = (asc == is_low) == self_first
            key = jnp.where(keep, key, pk)
            idx = jnp.where(keep, idx, pi)
            j //= 2
        k *= 2
    o_ref[...] = idx[:, :CAP]


def kernel(inputs, W, b):
    xs = inputs.reshape(B, S, G, 8).transpose(3, 0, 1, 2).astype(jnp.bfloat16)
    ws = W[:, 0].astype(jnp.bfloat16).reshape(G, 8).T

    logits = pl.pallas_call(
        _matvec_kernel,
        grid=(B, S // TB),
        in_specs=[
            pl.BlockSpec((8, 1, TB, G), lambda bi, si: (0, bi, si, 0)),
            pl.BlockSpec((8, G), lambda bi, si: (0, 0)),
        ],
        out_specs=pl.BlockSpec((1, 1, TB),
                               lambda bi, si: (bi * (S // TB) + si, 0, 0)),
        out_shape=jax.ShapeDtypeStruct((B * S // TB, 1, TB), jnp.float32),
    )(xs, ws)

    logits = logits.reshape(B, S) + b[0]

    return pl.pallas_call(
        _topk_kernel,
        out_shape=jax.ShapeDtypeStruct((B, CAP), jnp.int32),
    )(logits)


# speedup vs baseline: 1.0062x; 1.0062x over previous
import jax
import jax.numpy as jnp
import jax.experimental.pallas as pl
from jax.experimental.pallas import tpu as pltpu

B, S, D = 4, 8192, 2048
CAP = 2048
TB = 1024
G = D // 8
K_ALIGN = 27


def _matvec_kernel(xs_ref, ws_ref, o_ref):
    v4 = []
    for half in range(2):
        ps = []
        for i in range(4 * half, 4 * half + 4):
            a = xs_ref[i, 0].astype(jnp.float32)
            ps.append(a * ws_ref[i][None, :].astype(jnp.float32))
        amax = jnp.maximum(jnp.maximum(jnp.abs(ps[0]), jnp.abs(ps[1])),
                           jnp.maximum(jnp.abs(ps[2]), jnp.abs(ps[3])))
        bits = jax.lax.bitcast_convert_type(amax, jnp.int32)
        eb = jnp.clip(bits >> 23, 64, 254)
        graninv = jax.lax.bitcast_convert_type((281 - eb) << 23, jnp.float32)
        gran = jax.lax.bitcast_convert_type((eb - K_ALIGN) << 23, jnp.float32)
        s = ((ps[0] * graninv).astype(jnp.int32)
             + (ps[1] * graninv).astype(jnp.int32)
             + (ps[2] * graninv).astype(jnp.int32)
             + (ps[3] * graninv).astype(jnp.int32))
        v4.append(s.astype(jnp.float32) * gran)
    g8 = v4[0] + v4[1]

    acc = g8
    for k in range(1, 16):
        acc = acc + pltpu.roll(g8, G - k, 1)
    tot = acc
    for k in range(1, 16):
        tot = tot + pltpu.roll(acc, G - 16 * k, 1)
    o_ref[0, 0] = tot[:, 0]


def _topk_kernel(l_ref, o_ref):
    l = l_ref[...]
    m = jnp.max(l, axis=1, keepdims=True)
    key = jnp.exp(l - m)
    idx = jax.lax.broadcasted_iota(jnp.int32, (B, S), 1)
    lane = jax.lax.broadcasted_iota(jnp.int32, (B, S), 1)

    def rolled(a, j):
        return pltpu.roll(a, S - j, 1), pltpu.roll(a, j, 1)

    k = 2
    while k <= S:
        j = k // 2
        while j >= 1:
            is_low = (lane & j) == 0
            asc = (lane & k) == 0
            kl, kr = rolled(key, j)
            il, ir = rolled(idx, j)
            pk = jnp.where(is_low, kl, kr)
            pi = jnp.where(is_low, il, ir)
            self_first = (key > pk) | ((key == pk) & (idx < pi))
            keep = (asc == is_low) == self_first
            key = jnp.where(keep, key, pk)
            idx = jnp.where(keep, idx, pi)
            j //= 2
        k *= 2
    o_ref[...] = idx[:, :CAP]


def kernel(inputs, W, b):
    xs = inputs.reshape(B, S, G, 8).transpose(3, 0, 1, 2).astype(jnp.bfloat16)
    ws = W[:, 0].astype(jnp.bfloat16).reshape(G, 8).T

    logits = pl.pallas_call(
        _matvec_kernel,
        grid=(B, S // TB),
        in_specs=[
            pl.BlockSpec((8, 1, TB, G), lambda bi, si: (0, bi, si, 0)),
            pl.BlockSpec((8, G), lambda bi, si: (0, 0)),
        ],
        out_specs=pl.BlockSpec((1, 1, TB),
                               lambda bi, si: (bi * (S // TB) + si, 0, 0)),
        out_shape=jax.ShapeDtypeStruct((B * S // TB, 1, TB), jnp.float32),
    )(xs, ws)

    logits = logits.reshape(B, S) + b[0]

    return pl.pallas_call(
        _topk_kernel,
        out_shape=jax.ShapeDtypeStruct((B, CAP), jnp.int32),
    )(logits)
